# Initial kernel scaffold; baseline (speedup 1.0000x reference)
import functools

import jax
import jax.numpy as jnp
from jax import lax
from jax.experimental import pallas as pl
from jax.experimental.pallas import tpu as pltpu
from jax.experimental.pallas import tpu_sc as plsc

N = 320000
D = 128
NUM_SEG = 10000

NC = 2
NS = 16
NW = NC * NS
L = 16

CHUNK = 128
NCHUNKS = N // CHUNK
CW = 16
SEG_PER_TILE = NUM_SEG // NS
ZROWS = 125

_mesh = plsc.VectorSubcoreMesh(core_axis_name="c", subcore_axis_name="s")


@functools.partial(
    pl.kernel,
    out_type=(
        jax.ShapeDtypeStruct((NC, NUM_SEG, D), jnp.float32),
        jax.ShapeDtypeStruct((NC, NUM_SEG, CW), jnp.float32),
    ),
    mesh=_mesh,
    scratch_types=[
        pltpu.VMEM((CHUNK, D), jnp.float32),
        pltpu.VMEM((CHUNK,), jnp.int32),
        pltpu.VMEM((CHUNK, CW), jnp.float32),
        pltpu.VMEM((ZROWS, D), jnp.float32),
        pltpu.VMEM((SEG_PER_TILE, CW), jnp.float32),
        pltpu.VMEM_SHARED((NUM_SEG, D), jnp.float32),
        pltpu.VMEM_SHARED((NUM_SEG, CW), jnp.float32),
    ],
)
def _segsum_sc(x_hbm, batch_hbm, out_hbm, cnt_hbm,
               xb, ib, ones_v, zrow_v, zcnt_v, acc_sh, cnt_sh):
    cid = lax.axis_index("c")
    sid = lax.axis_index("s")
    wid = cid * NS + sid

    zeros16 = jnp.zeros((L,), jnp.float32)
    ones16 = jnp.ones((L,), jnp.float32)

    def fill_row(i, _):
        for j in range(D // L):
            zrow_v[i, pl.ds(j * L, L)] = zeros16
        return 0
    lax.fori_loop(0, ZROWS, fill_row, 0)

    def fill_cnt(i, _):
        zcnt_v[i, pl.ds(0, CW)] = zeros16
        return 0
    lax.fori_loop(0, SEG_PER_TILE, fill_cnt, 0)

    def fill_ones(i, _):
        ones_v[i, pl.ds(0, CW)] = ones16
        return 0
    lax.fori_loop(0, CHUNK, fill_ones, 0)

    sbase = sid * SEG_PER_TILE
    for k in range(SEG_PER_TILE // ZROWS):
        pltpu.sync_copy(zrow_v, acc_sh.at[pl.ds(sbase + k * ZROWS, ZROWS)])
    pltpu.sync_copy(zcnt_v, cnt_sh.at[pl.ds(sbase, SEG_PER_TILE)])
    plsc.subcore_barrier()

    nk = (NCHUNKS - wid + NW - 1) // NW

    def body(k, _):
        base = (wid + k * NW) * CHUNK
        pltpu.sync_copy(x_hbm.at[pl.ds(base, CHUNK)], xb)
        pltpu.sync_copy(batch_hbm.at[pl.ds(base, CHUNK)], ib)
        pltpu.sync_copy(xb, acc_sh.at[ib], add=True)
        pltpu.sync_copy(ones_v, cnt_sh.at[ib], add=True)
        return 0
    lax.fori_loop(0, nk, body, 0)

    plsc.subcore_barrier()

    pltpu.sync_copy(acc_sh.at[pl.ds(sbase, SEG_PER_TILE)],
                    out_hbm.at[cid, pl.ds(sbase, SEG_PER_TILE)])
    pltpu.sync_copy(cnt_sh.at[pl.ds(sbase, SEG_PER_TILE)],
                    cnt_hbm.at[cid, pl.ds(sbase, SEG_PER_TILE)])


BLK = 2000


def _mlp_body(s_ref, c_ref, win_ref, bin_ref, w1_ref, b1_ref, w2_ref, b2_ref,
              o_ref):
    s = s_ref[0] + s_ref[1]
    cnt = c_ref[0, :, 0:1] + c_ref[1, :, 0:1]
    pooled = jnp.dot(s, win_ref[...], preferred_element_type=jnp.float32)
    pooled = pooled + cnt * bin_ref[...]
    h = jnp.dot(pooled, w1_ref[...], preferred_element_type=jnp.float32)
    h = jnp.maximum(h + b1_ref[...], 0.0)
    o_ref[...] = (jnp.dot(h, w2_ref[...], preferred_element_type=jnp.float32)
                  + b2_ref[...])


def _mlp_tc(seg, cnt, W_in, b_in, W1, b1, W2, b2):
    grid = (NUM_SEG // BLK,)
    return pl.pallas_call(
        _mlp_body,
        grid=grid,
        in_specs=[
            pl.BlockSpec((NC, BLK, D), lambda i: (0, i, 0)),
            pl.BlockSpec((NC, BLK, CW), lambda i: (0, i, 0)),
            pl.BlockSpec((D, D), lambda i: (0, 0)),
            pl.BlockSpec((1, D), lambda i: (0, 0)),
            pl.BlockSpec((D, 2 * D), lambda i: (0, 0)),
            pl.BlockSpec((1, 2 * D), lambda i: (0, 0)),
            pl.BlockSpec((2 * D, D), lambda i: (0, 0)),
            pl.BlockSpec((1, D), lambda i: (0, 0)),
        ],
        out_specs=pl.BlockSpec((BLK, D), lambda i: (i, 0)),
        out_shape=jax.ShapeDtypeStruct((NUM_SEG, D), jnp.float32),
    )(seg, cnt, W_in, b_in.reshape(1, D), W1, b1.reshape(1, 2 * D),
      W2, b2.reshape(1, D))


def kernel(x, batch, W_in, b_in, W1, b1, W2, b2):
    batch = batch.astype(jnp.int32)
    seg, cnt = _segsum_sc(x, batch)
    return _mlp_tc(seg, cnt, W_in, b_in, W1, b1, W2, b2)


# speedup vs baseline: 1.5653x; 1.5653x over previous
import functools

import jax
import jax.numpy as jnp
from jax import lax
from jax.experimental import pallas as pl
from jax.experimental.pallas import tpu as pltpu
from jax.experimental.pallas import tpu_sc as plsc

N = 320000
D = 128
NUM_SEG = 10000
SEG_PAD = 10240

NC = 2
NS = 16
NW = NC * NS
L = 16

SPT = SEG_PAD // NW
RCHUNK = 128
NSEARCH = 19

_mesh = plsc.VectorSubcoreMesh(core_axis_name="c", subcore_axis_name="s")


@functools.partial(
    pl.kernel,
    out_type=jax.ShapeDtypeStruct((SEG_PAD, D), jnp.float32),
    mesh=_mesh,
    scratch_types=[
        pltpu.VMEM((RCHUNK, D), jnp.float32),
        pltpu.VMEM((RCHUNK + L,), jnp.int32),
        pltpu.VMEM((2 * L,), jnp.int32),
        pltpu.VMEM((SPT, D), jnp.float32),
    ],
)
def _segsum_sc(x_hbm, batch_hbm, out_hbm, xb, ib, sb, acc):
    cid = lax.axis_index("c")
    sid = lax.axis_index("s")
    t = sid * NC + cid
    s_lo = t * SPT

    zeros16 = jnp.zeros((L,), jnp.float32)

    def zero_body(i, _):
        for j in range(D // L):
            acc[i, pl.ds(j * L, L)] = zeros16
        return 0
    lax.fori_loop(0, SPT, zero_body, 0)

    def lower_bound(v):
        def it(_, lo_hi):
            lo, hi = lo_hi
            mid = (lo + hi) // 2
            base = jnp.minimum((mid // 8) * 8, N - 2 * L)
            pltpu.sync_copy(batch_hbm.at[pl.ds(base, 2 * L)], sb)
            val = sb[pl.ds(mid - base, L)][0]
            lt = val < v
            return (jnp.where(lt, mid + 1, lo), jnp.where(lt, hi, mid))
        lo, _ = lax.fori_loop(
            0, NSEARCH, it, (jnp.int32(0), jnp.int32(N))
        )
        return lo

    start = lower_bound(s_lo)
    end = lower_bound(s_lo + SPT)

    astart = (start // RCHUNK) * RCHUNK
    nchunks = (end - astart + RCHUNK - 1) // RCHUNK

    def chunk_body(k, _):
        base = astart + k * RCHUNK
        pltpu.sync_copy(x_hbm.at[pl.ds(base, RCHUNK)], xb)
        pltpu.sync_copy(batch_hbm.at[pl.ds(base, RCHUNK)],
                        ib.at[pl.ds(0, RCHUNK)])

        def row_body(r, _):
            sl = ib[pl.ds(r, L)][0] - s_lo

            @pl.when((sl >= 0) & (sl < SPT))
            def _():
                for j in range(D // L):
                    acc[sl, pl.ds(j * L, L)] = (
                        acc[sl, pl.ds(j * L, L)] + xb[r, pl.ds(j * L, L)]
                    )
            return 0
        lax.fori_loop(0, RCHUNK, row_body, 0)
        return 0
    lax.fori_loop(0, nchunks, chunk_body, 0)

    pltpu.sync_copy(acc, out_hbm.at[pl.ds(s_lo, SPT)])


IDR = 625
IDC = 512
IDB = 5
CROWS = SEG_PAD // 128


def _counts_body(ids_ref, o_ref):
    i = pl.program_id(0)

    @pl.when(i == 0)
    def _():
        o_ref[...] = jnp.zeros((CROWS, 128), jnp.float32)

    ids = ids_ref[0]
    first = ids[0, 0]
    last = ids[IDB - 1, IDC - 1]
    w0 = (first // 128) * 128
    nwin = (last - w0) // 128 + 1

    def win(k, _):
        base = w0 + k * 128
        vals = base + lax.broadcasted_iota(jnp.int32, (1, 1, 128), 2)
        eq = (ids[:, :, None] == vals).astype(jnp.float32)
        c = jnp.sum(eq, axis=(0, 1)).reshape(1, 128)
        o_ref[pl.ds(base // 128, 1), :] += c
        return 0
    lax.fori_loop(0, nwin, win, 0)


def _counts_tc(batch):
    ids = batch.reshape(IDR // IDB, IDB, IDC)
    grid = (IDR // IDB,)
    return pl.pallas_call(
        _counts_body,
        grid=grid,
        in_specs=[pl.BlockSpec((1, IDB, IDC), lambda i: (i, 0, 0))],
        out_specs=pl.BlockSpec((CROWS, 128), lambda i: (0, 0)),
        out_shape=jax.ShapeDtypeStruct((CROWS, 128), jnp.float32),
    )(ids)


BLK = 2048


def _mlp_body(s_ref, c_ref, win_ref, bin_ref, w1_ref, b1_ref, w2_ref, b2_ref,
              o_ref):
    s = s_ref[...]
    cnt = c_ref[...]
    pooled = jnp.dot(s, win_ref[...], preferred_element_type=jnp.float32)
    pooled = pooled + cnt * bin_ref[...]
    h = jnp.dot(pooled, w1_ref[...], preferred_element_type=jnp.float32)
    h = jnp.maximum(h + b1_ref[...], 0.0)
    o_ref[...] = (jnp.dot(h, w2_ref[...], preferred_element_type=jnp.float32)
                  + b2_ref[...])


def _mlp_tc(seg, cnt, W_in, b_in, W1, b1, W2, b2):
    grid = (SEG_PAD // BLK,)
    return pl.pallas_call(
        _mlp_body,
        grid=grid,
        in_specs=[
            pl.BlockSpec((BLK, D), lambda i: (i, 0)),
            pl.BlockSpec((BLK, 1), lambda i: (i, 0)),
            pl.BlockSpec((D, D), lambda i: (0, 0)),
            pl.BlockSpec((1, D), lambda i: (0, 0)),
            pl.BlockSpec((D, 2 * D), lambda i: (0, 0)),
            pl.BlockSpec((1, 2 * D), lambda i: (0, 0)),
            pl.BlockSpec((2 * D, D), lambda i: (0, 0)),
            pl.BlockSpec((1, D), lambda i: (0, 0)),
        ],
        out_specs=pl.BlockSpec((BLK, D), lambda i: (i, 0)),
        out_shape=jax.ShapeDtypeStruct((SEG_PAD, D), jnp.float32),
    )(seg, cnt, W_in, b_in.reshape(1, D), W1, b1.reshape(1, 2 * D),
      W2, b2.reshape(1, D))


def kernel(x, batch, W_in, b_in, W1, b1, W2, b2):
    batch = batch.astype(jnp.int32)
    seg = _segsum_sc(x, batch)
    cnt = _counts_tc(batch).reshape(SEG_PAD, 1)
    out = _mlp_tc(seg, cnt, W_in, b_in, W1, b1, W2, b2)
    return out[:NUM_SEG]


# speedup vs baseline: 1.6088x; 1.0278x over previous
import functools

import jax
import jax.numpy as jnp
from jax import lax
from jax.experimental import pallas as pl
from jax.experimental.pallas import tpu as pltpu
from jax.experimental.pallas import tpu_sc as plsc

N = 320000
D = 128
NUM_SEG = 10000
SEG_PAD = 10240

NC = 2
NS = 16
NW = NC * NS
L = 16

SPT = SEG_PAD // NW
RCHUNK = 512
NSEARCH = 19

_mesh = plsc.VectorSubcoreMesh(core_axis_name="c", subcore_axis_name="s")


@functools.partial(
    pl.kernel,
    out_type=jax.ShapeDtypeStruct((SEG_PAD, D), jnp.float32),
    mesh=_mesh,
    scratch_types=[
        pltpu.VMEM((RCHUNK, D), jnp.float32),
        pltpu.VMEM((RCHUNK + L,), jnp.int32),
        pltpu.VMEM((2 * L,), jnp.int32),
        pltpu.VMEM((SPT, D), jnp.float32),
    ],
)
def _segsum_sc(x_hbm, batch_hbm, out_hbm, xb, ib, sb, acc):
    cid = lax.axis_index("c")
    sid = lax.axis_index("s")
    t = sid * NC + cid
    s_lo = t * SPT

    zeros16 = jnp.zeros((L,), jnp.float32)

    def zero_body(i, _):
        for j in range(D // L):
            acc[i, pl.ds(j * L, L)] = zeros16
        return 0
    lax.fori_loop(0, SPT, zero_body, 0)

    def lower_bound(v):
        def it(_, lo_hi):
            lo, hi = lo_hi
            mid = (lo + hi) // 2
            base = jnp.minimum((mid // 8) * 8, N - 2 * L)
            pltpu.sync_copy(batch_hbm.at[pl.ds(base, 2 * L)], sb)
            val = sb[pl.ds(mid - base, L)][0]
            lt = val < v
            return (jnp.where(lt, mid + 1, lo), jnp.where(lt, hi, mid))
        lo, _ = lax.fori_loop(
            0, NSEARCH, it, (jnp.int32(0), jnp.int32(N))
        )
        return lo

    start = lower_bound(s_lo)
    end = lower_bound(s_lo + SPT)

    astart = (start // RCHUNK) * RCHUNK
    nchunks = (end - astart + RCHUNK - 1) // RCHUNK

    def chunk_body(k, _):
        base = astart + k * RCHUNK
        pltpu.sync_copy(x_hbm.at[pl.ds(base, RCHUNK)], xb)
        pltpu.sync_copy(batch_hbm.at[pl.ds(base, RCHUNK)],
                        ib.at[pl.ds(0, RCHUNK)])

        def row_body(r, _):
            sl = ib[pl.ds(r, L)][0] - s_lo

            @pl.when((sl >= 0) & (sl < SPT))
            def _():
                for j in range(D // L):
                    acc[sl, pl.ds(j * L, L)] = (
                        acc[sl, pl.ds(j * L, L)] + xb[r, pl.ds(j * L, L)]
                    )
            return 0
        lax.fori_loop(0, RCHUNK, row_body, 0)
        return 0
    lax.fori_loop(0, nchunks, chunk_body, 0)

    pltpu.sync_copy(acc, out_hbm.at[pl.ds(s_lo, SPT)])


IDR = 625
IDC = 512
IDB = 5
CROWS = SEG_PAD // 128


def _counts_body(ids_ref, o_ref):
    i = pl.program_id(0)

    @pl.when(i == 0)
    def _():
        o_ref[...] = jnp.zeros((CROWS, 128), jnp.float32)

    ids = ids_ref[0]
    first = ids[0, 0]
    last = ids[IDB - 1, IDC - 1]
    w0 = (first // 128) * 128
    nwin = (last - w0) // 128 + 1

    def win(k, _):
        base = w0 + k * 128
        vals = base + lax.broadcasted_iota(jnp.int32, (1, 1, 128), 2)
        eq = (ids[:, :, None] == vals).astype(jnp.float32)
        c = jnp.sum(eq, axis=(0, 1)).reshape(1, 128)
        o_ref[pl.ds(base // 128, 1), :] += c
        return 0
    lax.fori_loop(0, nwin, win, 0)


def _counts_tc(batch):
    ids = batch.reshape(IDR // IDB, IDB, IDC)
    grid = (IDR // IDB,)
    return pl.pallas_call(
        _counts_body,
        grid=grid,
        in_specs=[pl.BlockSpec((1, IDB, IDC), lambda i: (i, 0, 0))],
        out_specs=pl.BlockSpec((CROWS, 128), lambda i: (0, 0)),
        out_shape=jax.ShapeDtypeStruct((CROWS, 128), jnp.float32),
    )(ids)


BLK = 2048


def _mlp_body(s_ref, c_ref, win_ref, bin_ref, w1_ref, b1_ref, w2_ref, b2_ref,
              o_ref):
    s = s_ref[...]
    cnt = c_ref[...]
    pooled = jnp.dot(s, win_ref[...], preferred_element_type=jnp.float32)
    pooled = pooled + cnt * bin_ref[...]
    h = jnp.dot(pooled, w1_ref[...], preferred_element_type=jnp.float32)
    h = jnp.maximum(h + b1_ref[...], 0.0)
    o_ref[...] = (jnp.dot(h, w2_ref[...], preferred_element_type=jnp.float32)
                  + b2_ref[...])


def _mlp_tc(seg, cnt, W_in, b_in, W1, b1, W2, b2):
    grid = (SEG_PAD // BLK,)
    return pl.pallas_call(
        _mlp_body,
        grid=grid,
        in_specs=[
            pl.BlockSpec((BLK, D), lambda i: (i, 0)),
            pl.BlockSpec((BLK, 1), lambda i: (i, 0)),
            pl.BlockSpec((D, D), lambda i: (0, 0)),
            pl.BlockSpec((1, D), lambda i: (0, 0)),
            pl.BlockSpec((D, 2 * D), lambda i: (0, 0)),
            pl.BlockSpec((1, 2 * D), lambda i: (0, 0)),
            pl.BlockSpec((2 * D, D), lambda i: (0, 0)),
            pl.BlockSpec((1, D), lambda i: (0, 0)),
        ],
        out_specs=pl.BlockSpec((BLK, D), lambda i: (i, 0)),
        out_shape=jax.ShapeDtypeStruct((SEG_PAD, D), jnp.float32),
    )(seg, cnt, W_in, b_in.reshape(1, D), W1, b1.reshape(1, 2 * D),
      W2, b2.reshape(1, D))


def kernel(x, batch, W_in, b_in, W1, b1, W2, b2):
    batch = batch.astype(jnp.int32)
    seg = _segsum_sc(x, batch)
    cnt = _counts_tc(batch).reshape(SEG_PAD, 1)
    out = _mlp_tc(seg, cnt, W_in, b_in, W1, b1, W2, b2)
    return out[:NUM_SEG]


# speedup vs baseline: 2.0610x; 1.2811x over previous
import functools

import jax
import jax.numpy as jnp
from jax import lax
from jax.experimental import pallas as pl
from jax.experimental.pallas import tpu as pltpu
from jax.experimental.pallas import tpu_sc as plsc

N = 320000
D = 128
NUM_SEG = 10000
SEG_PAD = 10240

NC = 2
NS = 16
NW = NC * NS
L = 16

SPT = SEG_PAD // NW
RCHUNK = 512
NSEARCH = 19

_mesh = plsc.VectorSubcoreMesh(core_axis_name="c", subcore_axis_name="s")


@functools.partial(
    pl.kernel,
    out_type=jax.ShapeDtypeStruct((SEG_PAD, D), jnp.float32),
    mesh=_mesh,
    scratch_types=[
        pltpu.VMEM((RCHUNK, D), jnp.float32),
        pltpu.VMEM((RCHUNK + L,), jnp.int32),
        pltpu.VMEM((2 * L,), jnp.int32),
        pltpu.VMEM((SPT, D), jnp.float32),
    ],
)
def _segsum_sc(x_hbm, batch_hbm, out_hbm, xb, ib, sb, acc):
    cid = lax.axis_index("c")
    sid = lax.axis_index("s")
    t = sid * NC + cid
    s_lo = t * SPT

    zeros16 = jnp.zeros((L,), jnp.float32)

    def zero_body(i, _):
        for j in range(D // L):
            acc[i, pl.ds(j * L, L)] = zeros16
        return 0
    lax.fori_loop(0, SPT, zero_body, 0)

    def lower_bound(v):
        def it(_, lo_hi):
            lo, hi = lo_hi
            mid = (lo + hi) // 2
            base = jnp.minimum((mid // 8) * 8, N - 2 * L)
            pltpu.sync_copy(batch_hbm.at[pl.ds(base, 2 * L)], sb)
            val = sb[pl.ds(mid - base, L)][0]
            lt = val < v
            return (jnp.where(lt, mid + 1, lo), jnp.where(lt, hi, mid))
        lo, _ = lax.fori_loop(
            0, NSEARCH, it, (jnp.int32(0), jnp.int32(N))
        )
        return lo

    start = lower_bound(s_lo)
    end = lower_bound(s_lo + SPT)

    astart = (start // RCHUNK) * RCHUNK
    nchunks = (end - astart + RCHUNK - 1) // RCHUNK

    def chunk_body(k, _):
        base = astart + k * RCHUNK
        pltpu.sync_copy(x_hbm.at[pl.ds(base, RCHUNK)], xb)
        pltpu.sync_copy(batch_hbm.at[pl.ds(base, RCHUNK)],
                        ib.at[pl.ds(0, RCHUNK)])

        def grp_body(g, _):
            r0 = g * L
            ids16 = ib[pl.ds(r0, L)] - s_lo
            for rr in range(L):
                sl = ids16[rr]

                @pl.when((sl >= 0) & (sl < SPT))
                def _():
                    for j in range(D // L):
                        acc[sl, pl.ds(j * L, L)] = (
                            acc[sl, pl.ds(j * L, L)]
                            + xb[r0 + rr, pl.ds(j * L, L)]
                        )
            return 0
        lax.fori_loop(0, RCHUNK // L, grp_body, 0)
        return 0
    lax.fori_loop(0, nchunks, chunk_body, 0)

    pltpu.sync_copy(acc, out_hbm.at[pl.ds(s_lo, SPT)])


IDR = 625
IDC = 512
IDB = 5
CROWS = SEG_PAD // 128


def _counts_body(ids_ref, o_ref):
    i = pl.program_id(0)

    @pl.when(i == 0)
    def _():
        o_ref[...] = jnp.zeros((CROWS, 128), jnp.float32)

    ids = ids_ref[0]
    first = ids[0, 0]
    last = ids[IDB - 1, IDC - 1]
    w0 = (first // 128) * 128
    nwin = (last - w0) // 128 + 1

    def win(k, _):
        base = w0 + k * 128
        vals = base + lax.broadcasted_iota(jnp.int32, (1, 1, 128), 2)
        eq = (ids[:, :, None] == vals).astype(jnp.float32)
        c = jnp.sum(eq, axis=(0, 1)).reshape(1, 128)
        o_ref[pl.ds(base // 128, 1), :] += c
        return 0
    lax.fori_loop(0, nwin, win, 0)


def _counts_tc(batch):
    ids = batch.reshape(IDR // IDB, IDB, IDC)
    grid = (IDR // IDB,)
    return pl.pallas_call(
        _counts_body,
        grid=grid,
        in_specs=[pl.BlockSpec((1, IDB, IDC), lambda i: (i, 0, 0))],
        out_specs=pl.BlockSpec((CROWS, 128), lambda i: (0, 0)),
        out_shape=jax.ShapeDtypeStruct((CROWS, 128), jnp.float32),
    )(ids)


BLK = 2048


def _mlp_body(s_ref, c_ref, win_ref, bin_ref, w1_ref, b1_ref, w2_ref, b2_ref,
              o_ref):
    s = s_ref[...]
    cnt = c_ref[...]
    pooled = jnp.dot(s, win_ref[...], preferred_element_type=jnp.float32)
    pooled = pooled + cnt * bin_ref[...]
    h = jnp.dot(pooled, w1_ref[...], preferred_element_type=jnp.float32)
    h = jnp.maximum(h + b1_ref[...], 0.0)
    o_ref[...] = (jnp.dot(h, w2_ref[...], preferred_element_type=jnp.float32)
                  + b2_ref[...])


def _mlp_tc(seg, cnt, W_in, b_in, W1, b1, W2, b2):
    grid = (SEG_PAD // BLK,)
    return pl.pallas_call(
        _mlp_body,
        grid=grid,
        in_specs=[
            pl.BlockSpec((BLK, D), lambda i: (i, 0)),
            pl.BlockSpec((BLK, 1), lambda i: (i, 0)),
            pl.BlockSpec((D, D), lambda i: (0, 0)),
            pl.BlockSpec((1, D), lambda i: (0, 0)),
            pl.BlockSpec((D, 2 * D), lambda i: (0, 0)),
            pl.BlockSpec((1, 2 * D), lambda i: (0, 0)),
            pl.BlockSpec((2 * D, D), lambda i: (0, 0)),
            pl.BlockSpec((1, D), lambda i: (0, 0)),
        ],
        out_specs=pl.BlockSpec((BLK, D), lambda i: (i, 0)),
        out_shape=jax.ShapeDtypeStruct((SEG_PAD, D), jnp.float32),
    )(seg, cnt, W_in, b_in.reshape(1, D), W1, b1.reshape(1, 2 * D),
      W2, b2.reshape(1, D))


def kernel(x, batch, W_in, b_in, W1, b1, W2, b2):
    batch = batch.astype(jnp.int32)
    seg = _segsum_sc(x, batch)
    cnt = _counts_tc(batch).reshape(SEG_PAD, 1)
    out = _mlp_tc(seg, cnt, W_in, b_in, W1, b1, W2, b2)
    return out[:NUM_SEG]
